# baseline (speedup 1.0000x reference)
import functools

import jax
import jax.numpy as jnp
from jax import lax
from jax.experimental import pallas as pl
from jax.experimental.pallas import tpu as pltpu
from jax.experimental.pallas import tpu_sc as plsc

_B, _N, _D, _E = 4, 10000, 128, 320000
_NS = 16
_NP = 10240
_RPT = _NP // _NS
_K = 64
_CPT = 320
_EPT = _CPT * _K
_EPAD = _EPT * _NS
_CPS = _EPAD // _K
_NB = 1000
_DEPTH = 4


def _mm_body(x_ref, w_ref, o_ref):
    o_ref[0, 0] = jnp.dot(x_ref[0], w_ref[0],
                          preferred_element_type=jnp.float32)


def _matmul(x, ws):
    return pl.pallas_call(
        _mm_body,
        grid=(2, _B, _N // _NB),
        in_specs=[
            pl.BlockSpec((1, _NB, _D), lambda s, b, n: (b, n, 0)),
            pl.BlockSpec((1, _D, _D), lambda s, b, n: (s, 0, 0)),
        ],
        out_specs=pl.BlockSpec((1, 1, _NB, _D), lambda s, b, n: (s, b, n, 0)),
        out_shape=jax.ShapeDtypeStruct((2, _B, _N, _D), jnp.float32),
    )(x, ws)


def _sc_body(p_hbm, ed_hbm, ew_hbm, out_hbm, *scr):
    rows = scr[0:2]
    sbuf = scr[2:4]
    ed = scr[4:6]
    eww = scr[6:8]
    sidx = scr[8:10]
    esem = scr[10:12]
    gsem = scr[12:14]
    ssem = scr[14:16]
    accum = scr[16]

    cid = lax.axis_index("c")
    sid = lax.axis_index("s")
    row0 = sid * _RPT
    gid0 = cid * _CPS + sid * _CPT
    z16 = jnp.zeros((16,), jnp.float32)
    jidx = [jnp.full((16,), j, jnp.int32) for j in range(16)]

    def e_start(j, q):
        pltpu.make_async_copy(ed_hbm.at[gid0 + j], ed[q], esem[q]).start()
        pltpu.make_async_copy(ew_hbm.at[gid0 + j], eww[q], esem[q]).start()

    def e_wait(q):
        pltpu.make_async_copy(ed_hbm.at[gid0], ed[q], esem[q]).wait()
        pltpu.make_async_copy(ew_hbm.at[gid0], eww[q], esem[q]).wait()

    def g_start(b, q):
        pltpu.make_async_copy(
            p_hbm.at[cid, b].at[ed[q].at[0]], rows[q], gsem[q]).start()

    def g_wait(b, q):
        pltpu.make_async_copy(
            p_hbm.at[cid, b].at[ed[q].at[0]], rows[q], gsem[q]).wait()

    def s_start(q):
        pltpu.async_copy(sbuf[q], accum.at[sidx[q]], ssem[q], add=True)

    def s_wait(q):
        pltpu.make_async_copy(sbuf[q], accum.at[sidx[q]], ssem[q]).wait()

    def _scale(q):
        def body(bk, carry):
            ew16 = eww[q][0, pl.ds(bk * 16, 16)]
            for j in range(16):
                ewb = ew16.at[jidx[j]].get(mode="promise_in_bounds")
                i = bk * 16 + j
                for qq in range(_D // 16):
                    sl = pl.ds(qq * 16, 16)
                    sbuf[q][i, sl] = rows[q][i, sl] * ewb
            return carry
        lax.fori_loop(0, _K // 16, body, 0)

    def batch(b, carry):
        def zrow(r, c2):
            for qq in range(_D // 16):
                rows[0][r, pl.ds(qq * 16, 16)] = z16
            return c2
        lax.fori_loop(0, _K, zrow, 0)
        for off in range(0, _RPT, _K):
            pltpu.sync_copy(rows[0], accum.at[pl.ds(row0 + off, _K)])
        plsc.subcore_barrier()

        e_start(0, 0)
        e_start(1, 1)
        e_wait(0)
        g_start(b, 0)

        def group(g, c2):
            for q in range(2):
                j = g * 2 + q

                @pl.when(j >= 2)
                def _():
                    s_wait(q)

                g_wait(b, q)
                for t in range(_K // 16):
                    sl = pl.ds(t * 16, 16)
                    sidx[q][sl] = ed[q][1, sl]
                _scale(q)
                s_start(q)

                @pl.when(j + 2 < _CPT)
                def _():
                    e_start(j + 2, q)

                @pl.when(j + 1 < _CPT)
                def _():
                    e_wait(1 - q)
                    g_start(b, 1 - q)
            return c2
        lax.fori_loop(0, _CPT // 2, group, 0)
        s_wait(0)
        s_wait(1)
        plsc.subcore_barrier()

        for off in range(0, _RPT, _K):
            pltpu.sync_copy(accum.at[pl.ds(row0 + off, _K)], rows[0])
            pltpu.sync_copy(rows[0], out_hbm.at[cid, b, pl.ds(row0 + off, _K)])
        plsc.subcore_barrier()
        return carry

    lax.fori_loop(0, _B, batch, 0)


_sc_spmm = functools.partial(
    pl.kernel,
    out_type=jax.ShapeDtypeStruct((2, _B, _NP, _D), jnp.float32),
    mesh=plsc.VectorSubcoreMesh(core_axis_name="c", subcore_axis_name="s"),
    scratch_types=(
        [pltpu.VMEM((_K, _D), jnp.float32) for _ in range(4)]
        + [pltpu.VMEM((2, _K), jnp.int32) for _ in range(2)]
        + [pltpu.VMEM((1, _K), jnp.float32) for _ in range(2)]
        + [pltpu.VMEM((_K,), jnp.int32) for _ in range(2)]
        + [pltpu.SemaphoreType.DMA for _ in range(6)]
        + [pltpu.VMEM_SHARED((_NP, _D), jnp.float32)]
    ),
)(_sc_body)


def kernel(inputs, edge_index0, edge_weight0, edge_index1, edge_weight1,
           W0, W1):
    ws = jnp.stack([W0, W1])
    p = _matmul(inputs, ws)
    zpad_i = jnp.zeros((_EPAD - _E,), jnp.int32)
    zpad_f = jnp.zeros((_EPAD - _E,), jnp.float32)
    src = jnp.concatenate([edge_index0[1], zpad_i, edge_index1[1], zpad_i])
    dst = jnp.concatenate([edge_index0[0], zpad_i, edge_index1[0], zpad_i])
    ew = jnp.concatenate([edge_weight0, zpad_f, edge_weight1, zpad_f])
    edata = jnp.stack([src.reshape(-1, _K), dst.reshape(-1, _K)], axis=1)
    res = _sc_spmm(p, edata, ew.reshape(-1, 1, _K))
    return jnp.concatenate([res[0, :, :_N], res[1, :, :_N]], axis=-1)


# speedup vs baseline: 1.5054x; 1.5054x over previous
import functools

import jax
import jax.numpy as jnp
from jax import lax
from jax.experimental import pallas as pl
from jax.experimental.pallas import tpu as pltpu
from jax.experimental.pallas import tpu_sc as plsc

_B, _N, _D, _E = 4, 10000, 128, 320000
_NS = 16
_NP = 10240
_RPT = _NP // _NS
_K = 80
_CPT = 256
_EPT = _CPT * _K
_EPAD = _EPT * _NS
_CPS = _EPAD // _K
_NB = 1000
_DEPTH = 4


def _mm_body(x_ref, w_ref, o_ref):
    o_ref[0, 0] = jnp.dot(x_ref[0], w_ref[0],
                          preferred_element_type=jnp.float32)


def _matmul(x, ws):
    return pl.pallas_call(
        _mm_body,
        grid=(2, _B, _N // _NB),
        in_specs=[
            pl.BlockSpec((1, _NB, _D), lambda s, b, n: (b, n, 0)),
            pl.BlockSpec((1, _D, _D), lambda s, b, n: (s, 0, 0)),
        ],
        out_specs=pl.BlockSpec((1, 1, _NB, _D), lambda s, b, n: (s, b, n, 0)),
        out_shape=jax.ShapeDtypeStruct((2, _B, _N, _D), jnp.float32),
    )(x, ws)


def _sc_body(p_hbm, ed_hbm, ew_hbm, out_hbm, *scr):
    rows = scr[0:4]
    ed = scr[4:8]
    eww = scr[8:12]
    esem = scr[12:16]
    gsem = scr[16:20]
    ssem = scr[20:24]
    accum = scr[24]

    cid = lax.axis_index("c")
    sid = lax.axis_index("s")
    row0 = sid * _RPT
    gid0 = cid * _CPS + sid * _CPT
    z16 = jnp.zeros((16,), jnp.float32)
    jidx = [jnp.full((16,), j, jnp.int32) for j in range(16)]

    def e_start(j, q):
        pltpu.make_async_copy(ed_hbm.at[gid0 + j], ed[q], esem[q]).start()
        pltpu.make_async_copy(ew_hbm.at[gid0 + j], eww[q], esem[q]).start()

    def e_wait(q):
        pltpu.make_async_copy(ed_hbm.at[gid0], ed[q], esem[q]).wait()
        pltpu.make_async_copy(ew_hbm.at[gid0], eww[q], esem[q]).wait()

    def g_start(b, q):
        pltpu.make_async_copy(
            p_hbm.at[cid, b].at[ed[q].at[0]], rows[q], gsem[q]).start()

    def g_wait(b, q):
        pltpu.make_async_copy(
            p_hbm.at[cid, b].at[ed[q].at[0]], rows[q], gsem[q]).wait()

    def s_start(q):
        pltpu.async_copy(rows[q], accum.at[ed[q].at[1]], ssem[q], add=True)

    def s_wait(q):
        pltpu.make_async_copy(rows[q], accum.at[ed[q].at[1]],
                              ssem[q]).wait()

    def _scale(q):
        def body(bk, carry):
            ew16 = eww[q][0, pl.ds(bk * 16, 16)]
            for j in range(16):
                ewb = ew16.at[jidx[j]].get(mode="promise_in_bounds")
                i = bk * 16 + j
                for qq in range(_D // 16):
                    sl = pl.ds(qq * 16, 16)
                    rows[q][i, sl] = rows[q][i, sl] * ewb
            return carry
        lax.fori_loop(0, _K // 16, body, 0)

    def batch(b, carry):
        def zrow(r, c2):
            for qq in range(_D // 16):
                rows[0][r, pl.ds(qq * 16, 16)] = z16
            return c2
        lax.fori_loop(0, _K, zrow, 0)
        for off in range(0, _RPT, _K):
            pltpu.sync_copy(rows[0], accum.at[pl.ds(row0 + off, _K)])
        plsc.subcore_barrier()

        e_start(0, 0)
        e_start(1, 1)
        e_start(2, 2)
        e_wait(0)
        g_start(b, 0)
        e_wait(1)
        g_start(b, 1)

        def group(g, c2):
            for par in range(4):
                j = g * 4 + par
                p2, p3 = (par + 2) % 4, (par + 3) % 4

                g_wait(b, par)

                @pl.when(j >= 1)
                def _():
                    s_wait(p3)

                _scale(par)
                s_start(par)

                @pl.when(j + 3 < _CPT)
                def _():
                    e_start(j + 3, p3)

                @pl.when(j + 2 < _CPT)
                def _():
                    e_wait(p2)
                    g_start(b, p2)
            return c2
        lax.fori_loop(0, _CPT // 4, group, 0)
        s_wait((_CPT - 1) % 4)
        plsc.subcore_barrier()

        for off in range(0, _RPT, _K):
            pltpu.sync_copy(accum.at[pl.ds(row0 + off, _K)], rows[0])
            pltpu.sync_copy(rows[0], out_hbm.at[cid, b, pl.ds(row0 + off, _K)])
        plsc.subcore_barrier()
        return carry

    lax.fori_loop(0, _B, batch, 0)


_sc_spmm = functools.partial(
    pl.kernel,
    out_type=jax.ShapeDtypeStruct((2, _B, _NP, _D), jnp.float32),
    mesh=plsc.VectorSubcoreMesh(core_axis_name="c", subcore_axis_name="s"),
    scratch_types=(
        [pltpu.VMEM((_K, _D), jnp.float32) for _ in range(4)]
        + [pltpu.VMEM((2, _K), jnp.int32) for _ in range(4)]
        + [pltpu.VMEM((1, _K), jnp.float32) for _ in range(4)]
        + [pltpu.SemaphoreType.DMA for _ in range(12)]
        + [pltpu.VMEM_SHARED((_NP, _D), jnp.float32)]
    ),
)(_sc_body)


def kernel(inputs, edge_index0, edge_weight0, edge_index1, edge_weight1,
           W0, W1):
    ws = jnp.stack([W0, W1])
    p = _matmul(inputs, ws)
    zpad_i = jnp.zeros((_EPAD - _E,), jnp.int32)
    zpad_f = jnp.zeros((_EPAD - _E,), jnp.float32)
    src = jnp.concatenate([edge_index0[1], zpad_i, edge_index1[1], zpad_i])
    dst = jnp.concatenate([edge_index0[0], zpad_i, edge_index1[0], zpad_i])
    ew = jnp.concatenate([edge_weight0, zpad_f, edge_weight1, zpad_f])
    edata = jnp.stack([src.reshape(-1, _K), dst.reshape(-1, _K)], axis=1)
    res = _sc_spmm(p, edata, ew.reshape(-1, 1, _K))
    return jnp.concatenate([res[0, :, :_N], res[1, :, :_N]], axis=-1)
